# baseline (speedup 1.0000x reference)
import functools

import jax
import jax.numpy as jnp
from jax import lax
from jax.experimental import pallas as pl
from jax.experimental.pallas import tpu as pltpu
from jax.experimental.pallas import tpu_sc as plsc

_HIDDEN = 128
_EPS = 1e-12
_SEQ = 20
_PADSEQ = 24
_EPW = 4
_WINDOW = _EPW * _SEQ


def _sc_gather3(wl_table, pos_table, hop_table, wl_i, pos_i, hop_i, batch):
    n = wl_i.shape[0]
    d = wl_table.shape[1]
    mesh = plsc.VectorSubcoreMesh(core_axis_name="c", subcore_axis_name="s")
    n_workers = mesh.num_cores * mesh.num_subcores
    rows_per_w = n // n_workers
    elems_per_w = batch // n_workers
    n_win = elems_per_w // _EPW
    assert rows_per_w % _WINDOW == 0 and elems_per_w % _EPW == 0

    out_sds = jax.ShapeDtypeStruct((batch * _PADSEQ, d), jnp.float32)

    vmem_idx = pltpu.VMEM((_WINDOW,), jnp.int32)
    vmem_rows = pltpu.VMEM((_WINDOW, d), jnp.float32)
    vmem_acc = pltpu.VMEM((_WINDOW + _PADSEQ, d), jnp.float32)

    @functools.partial(
        pl.kernel,
        out_type=out_sds,
        mesh=mesh,
        scratch_types=[
            vmem_idx, vmem_idx, vmem_idx, vmem_idx, vmem_idx, vmem_idx,
            vmem_acc, vmem_rows, vmem_rows, vmem_acc, vmem_rows, vmem_rows,
            pltpu.SemaphoreType.DMA, pltpu.SemaphoreType.DMA,
            pltpu.SemaphoreType.DMA, pltpu.SemaphoreType.DMA,
            pltpu.SemaphoreType.DMA, pltpu.SemaphoreType.DMA,
        ],
    )
    def sck(wl_t, pos_t, hop_t, wl_idx, pos_idx, hop_idx, osum,
            i00, i01, i02, i10, i11, i12,
            g00, g01, g02, g10, g11, g12,
            semi0, semi1, semg0, semg1, semo0, semo1):
        tbl = (wl_t, pos_t, hop_t)
        idx = (wl_idx, pos_idx, hop_idx)
        ic = ((i00, i01, i02), (i10, i11, i12))
        gb = ((g00, g01, g02), (g10, g11, g12))
        semi = (semi0, semi1)
        semg = (semg0, semg1)
        semo = (semo0, semo1)

        wid = lax.axis_index("s") * mesh.num_cores + lax.axis_index("c")
        base = wid * rows_per_w
        ebase = wid * elems_per_w

        def fire_idx(w, p):
            off = base + w * _WINDOW
            for k in range(3):
                pltpu.async_copy(idx[k].at[pl.ds(off, _WINDOW)], ic[p][k],
                                 semi[p])

        def drain_idx(p):
            for k in range(3):
                pltpu.make_async_copy(idx[k].at[pl.ds(base, _WINDOW)],
                                      ic[p][k], semi[p]).wait()

        def fire_gathers(p):
            for k in range(3):
                dst = gb[p][k].at[pl.ds(0, _WINDOW)] if k == 0 else gb[p][k]
                pltpu.async_copy(tbl[k].at[ic[p][k]], dst, semg[p])

        def drain_gathers(p):
            for k in range(3):
                dst = gb[p][k].at[pl.ds(0, _WINDOW)] if k == 0 else gb[p][k]
                pltpu.make_async_copy(tbl[k].at[pl.ds(0, _WINDOW)],
                                      dst, semg[p]).wait()

        def sum_bufs(p):
            a0, a1, a2 = gb[p]

            @pl.loop(0, _WINDOW)
            def _(r):
                for c in range(0, d, 16):
                    slc = (pl.ds(r, 1), pl.ds(c, 16))
                    a0.at[slc][...] = (
                        a0.at[slc][...] + a1.at[slc][...] + a2.at[slc][...])

        def fire_outputs(w, p):
            for e in range(_EPW):
                dst = (ebase + w * _EPW + e) * _PADSEQ
                pltpu.async_copy(gb[p][0].at[pl.ds(e * _SEQ, _PADSEQ)],
                                 osum.at[pl.ds(dst, _PADSEQ)], semo[p])

        def drain_outputs(p):
            for e in range(_EPW):
                pltpu.make_async_copy(tbl[0].at[pl.ds(0, _PADSEQ)],
                                      gb[p][0].at[pl.ds(0, _PADSEQ)],
                                      semo[p]).wait()

        fire_idx(0, 0)
        drain_idx(0)
        fire_gathers(0)
        fire_idx(1, 1)

        @pl.loop(0, n_win // 2)
        def _(j):
            for b in (0, 1):
                w = 2 * j + b
                nb = 1 - b

                @pl.when(w >= 1)
                def _():
                    drain_outputs(nb)

                @pl.when(w + 1 < n_win)
                def _():
                    drain_idx(nb)
                    fire_gathers(nb)

                drain_gathers(b)

                @pl.when(w + 2 < n_win)
                def _():
                    fire_idx(w + 2, b)

                sum_bufs(b)
                fire_outputs(w, b)

        drain_outputs((n_win - 1) % 2)

    return sck(wl_table, pos_table, hop_table, wl_i, pos_i, hop_i)


def _tc_body(raw_ref, g_ref, w_ref, b_ref, gamma_ref, beta_ref, o_ref):
    bt, s, d = raw_ref.shape
    sp = g_ref.shape[1]
    x3 = raw_ref[...]
    x3p = jnp.concatenate(
        [x3, jnp.zeros((bt, sp - s, d), jnp.float32)], axis=1)
    x = jnp.dot(x3p.reshape(bt * sp, d), w_ref[...],
                preferred_element_type=jnp.float32)
    e = x + b_ref[...] + g_ref[...].reshape(bt * sp, d)
    mean = jnp.mean(e, axis=-1, keepdims=True)
    c = e - mean
    var = jnp.mean(c * c, axis=-1, keepdims=True)
    o = c * lax.rsqrt(var + _EPS) * gamma_ref[...] + beta_ref[...]
    o_ref[...] = o.reshape(bt, sp, d)[:, :s, :]


def _tc_fuse(raw, g, w, b, gamma, beta, tile_b):
    batch, s, d = raw.shape
    grid = (batch // tile_b,)
    raw_spec = pl.BlockSpec((tile_b, s, d), lambda i: (i, 0, 0))
    g_spec = pl.BlockSpec((tile_b, _PADSEQ, d), lambda i: (i, 0, 0))
    full_spec = pl.BlockSpec((d, d), lambda i: (0, 0))
    vec_spec = pl.BlockSpec((1, d), lambda i: (0, 0))
    return pl.pallas_call(
        _tc_body,
        grid=grid,
        in_specs=[raw_spec, g_spec, full_spec,
                  vec_spec, vec_spec, vec_spec],
        out_specs=raw_spec,
        out_shape=jax.ShapeDtypeStruct((batch, s, d), jnp.float32),
        compiler_params=pltpu.CompilerParams(
            dimension_semantics=("parallel",)),
    )(raw, g, w, b.reshape(1, d), gamma.reshape(1, d), beta.reshape(1, d))


def kernel(raw_features, wl_role_ids, init_pos_ids, hop_dis_ids, W, b,
           wl_table, pos_table, hop_table, gamma, beta):
    batch, seq, x_size = raw_features.shape
    wl_i = wl_role_ids.reshape(-1).astype(jnp.int32)
    pos_i = init_pos_ids.reshape(-1).astype(jnp.int32)
    hop_i = hop_dis_ids.reshape(-1).astype(jnp.int32)

    g = _sc_gather3(wl_table, pos_table, hop_table, wl_i, pos_i, hop_i,
                    batch)
    g = g.reshape(batch, _PADSEQ, _HIDDEN)
    return _tc_fuse(raw_features, g, W, b, gamma, beta, tile_b=512)


# speedup vs baseline: 1.0573x; 1.0573x over previous
import functools

import jax
import jax.numpy as jnp
from jax import lax
from jax.experimental import pallas as pl
from jax.experimental.pallas import tpu as pltpu
from jax.experimental.pallas import tpu_sc as plsc

_HIDDEN = 128
_EPS = 1e-12
_WINDOW = 128
_CHUNKS = 2


def _sc_gathersum(wl_table, pos_table, hop_table, wl_i, pos_i, hop_i):
    n = wl_i.shape[0]
    d = wl_table.shape[1]
    mesh = plsc.VectorSubcoreMesh(core_axis_name="c", subcore_axis_name="s")
    n_workers = mesh.num_cores * mesh.num_subcores
    rows_per_w = n // n_workers
    n_win = rows_per_w // _WINDOW
    assert rows_per_w % _WINDOW == 0 and n_win % 2 == 0

    out_sds = jax.ShapeDtypeStruct((n, d), jnp.float32)

    vmem_idx = pltpu.VMEM((_WINDOW,), jnp.int32)
    vmem_rows = pltpu.VMEM((_WINDOW, d), jnp.float32)

    @functools.partial(
        pl.kernel,
        out_type=out_sds,
        mesh=mesh,
        scratch_types=[
            vmem_idx, vmem_idx, vmem_idx, vmem_idx, vmem_idx, vmem_idx,
            vmem_rows, vmem_rows, vmem_rows, vmem_rows, vmem_rows, vmem_rows,
            pltpu.SemaphoreType.DMA, pltpu.SemaphoreType.DMA,
            pltpu.SemaphoreType.DMA, pltpu.SemaphoreType.DMA,
            pltpu.SemaphoreType.DMA, pltpu.SemaphoreType.DMA,
        ],
    )
    def sck(wl_t, pos_t, hop_t, wl_idx, pos_idx, hop_idx, osum,
            i00, i01, i02, i10, i11, i12,
            g00, g01, g02, g10, g11, g12,
            semi0, semi1, semg0, semg1, semo0, semo1):
        tbl = (wl_t, pos_t, hop_t)
        idx = (wl_idx, pos_idx, hop_idx)
        ic = ((i00, i01, i02), (i10, i11, i12))
        gb = ((g00, g01, g02), (g10, g11, g12))
        semi = (semi0, semi1)
        semg = (semg0, semg1)
        semo = (semo0, semo1)

        wid = lax.axis_index("s") * mesh.num_cores + lax.axis_index("c")
        base = wid * rows_per_w

        def fire_idx(w, p):
            off = base + w * _WINDOW
            for k in range(3):
                pltpu.async_copy(idx[k].at[pl.ds(off, _WINDOW)], ic[p][k],
                                 semi[p])

        def drain_idx(p):
            for k in range(3):
                pltpu.make_async_copy(idx[k].at[pl.ds(base, _WINDOW)],
                                      ic[p][k], semi[p]).wait()

        def fire_gathers(p):
            for k in range(3):
                pltpu.async_copy(tbl[k].at[ic[p][k]], gb[p][k], semg[p])

        def drain_gathers(p):
            for k in range(3):
                pltpu.make_async_copy(tbl[k].at[pl.ds(0, _WINDOW)],
                                      gb[p][k], semg[p]).wait()

        def sum_bufs(p):
            a0, a1, a2 = gb[p]

            @pl.loop(0, _WINDOW)
            def _(r):
                for c in range(0, d, 16):
                    slc = (pl.ds(r, 1), pl.ds(c, 16))
                    a0.at[slc][...] = (
                        a0.at[slc][...] + a1.at[slc][...] + a2.at[slc][...])

        def fire_outputs(w, p):
            off = base + w * _WINDOW
            pltpu.async_copy(gb[p][0], osum.at[pl.ds(off, _WINDOW)], semo[p])

        def drain_outputs(p):
            pltpu.make_async_copy(tbl[0].at[pl.ds(0, _WINDOW)],
                                  gb[p][0], semo[p]).wait()

        fire_idx(0, 0)
        drain_idx(0)
        fire_gathers(0)
        fire_idx(1, 1)

        @pl.loop(0, n_win // 2)
        def _(j):
            for b in (0, 1):
                w = 2 * j + b
                nb = 1 - b

                @pl.when(w >= 1)
                def _():
                    drain_outputs(nb)

                @pl.when(w + 1 < n_win)
                def _():
                    drain_idx(nb)
                    fire_gathers(nb)

                drain_gathers(b)

                @pl.when(w + 2 < n_win)
                def _():
                    fire_idx(w + 2, b)

                sum_bufs(b)
                fire_outputs(w, b)

        drain_outputs((n_win - 1) % 2)

    return sck(wl_table, pos_table, hop_table, wl_i, pos_i, hop_i)


def _tc_body(raw_ref, g_ref, w_ref, b_ref, gamma_ref, beta_ref, o_ref):
    bt, s, d = raw_ref.shape
    x2 = raw_ref[...].reshape(bt * s, d)
    x = jnp.dot(x2, w_ref[...], preferred_element_type=jnp.float32)
    e = x + b_ref[...] + g_ref[...]
    mean = jnp.mean(e, axis=-1, keepdims=True)
    c = e - mean
    var = jnp.mean(c * c, axis=-1, keepdims=True)
    o = c * lax.rsqrt(var + _EPS) * gamma_ref[...] + beta_ref[...]
    o_ref[...] = o.reshape(bt, s, d)


def _tc_body_alias(raw_ref, g_ref, w_ref, b_ref, gamma_ref, beta_ref,
                   prev_ref, o_ref):
    del prev_ref
    _tc_body(raw_ref, g_ref, w_ref, b_ref, gamma_ref, beta_ref, o_ref)


def _tc_fuse_chunk(raw, g, w, b, gamma, beta, tile_b, chunk, chunk_b, prev):
    batch, s, d = raw.shape
    grid = (chunk_b // tile_b,)
    off = chunk * (chunk_b // tile_b)
    raw_spec = pl.BlockSpec((tile_b, s, d), lambda i: (off + i, 0, 0))
    g_spec = pl.BlockSpec((tile_b * s, d), lambda i: (i, 0))
    full_spec = pl.BlockSpec((d, d), lambda i: (0, 0))
    vec_spec = pl.BlockSpec((1, d), lambda i: (0, 0))
    in_specs = [raw_spec, g_spec, full_spec, vec_spec, vec_spec, vec_spec]
    args = [raw, g, w, b.reshape(1, d), gamma.reshape(1, d),
            beta.reshape(1, d)]
    kwargs = {}
    body = _tc_body
    if prev is not None:
        in_specs.append(pl.BlockSpec((8, s, d), lambda i: (0, 0, 0)))
        args.append(prev)
        kwargs["input_output_aliases"] = {6: 0}
        body = _tc_body_alias
    return pl.pallas_call(
        body,
        grid=grid,
        in_specs=in_specs,
        out_specs=raw_spec,
        out_shape=jax.ShapeDtypeStruct((batch, s, d), jnp.float32),
        compiler_params=pltpu.CompilerParams(
            dimension_semantics=("parallel",)),
        **kwargs,
    )(*args)


def kernel(raw_features, wl_role_ids, init_pos_ids, hop_dis_ids, W, b,
           wl_table, pos_table, hop_table, gamma, beta):
    batch, seq, x_size = raw_features.shape
    chunk_b = batch // _CHUNKS

    gs = []
    for c in range(_CHUNKS):
        sl = slice(c * chunk_b, (c + 1) * chunk_b)
        wl_i = wl_role_ids[sl].reshape(-1).astype(jnp.int32)
        pos_i = init_pos_ids[sl].reshape(-1).astype(jnp.int32)
        hop_i = hop_dis_ids[sl].reshape(-1).astype(jnp.int32)
        gs.append(_sc_gathersum(wl_table, pos_table, hop_table,
                                wl_i, pos_i, hop_i))

    out = None
    for c in range(_CHUNKS):
        out = _tc_fuse_chunk(raw_features, gs[c], W, b, gamma, beta,
                             tile_b=512, chunk=c, chunk_b=chunk_b, prev=out)
    return out


# speedup vs baseline: 1.7110x; 1.6183x over previous
import functools

import jax
import jax.numpy as jnp
from jax import lax
from jax.experimental import pallas as pl
from jax.experimental.pallas import tpu as pltpu
from jax.experimental.pallas import tpu_sc as plsc

_HIDDEN = 128
_EPS = 1e-12
_WINDOW = 128
_CHUNKS = 2


def _sc_gathersum(wl_table, pos_table, hop_table, wl_i, pos_i, hop_i):
    n = wl_i.shape[0]
    d = wl_table.shape[1]
    mesh = plsc.VectorSubcoreMesh(core_axis_name="c", subcore_axis_name="s")
    n_workers = mesh.num_cores * mesh.num_subcores
    rows_per_w = n // n_workers
    n_win = rows_per_w // _WINDOW
    assert rows_per_w % _WINDOW == 0 and n_win % 2 == 0

    out_sds = jax.ShapeDtypeStruct((n, d), jnp.float32)

    vmem_idx = pltpu.VMEM((_WINDOW,), jnp.int32)
    vmem_rows = pltpu.VMEM((_WINDOW, d), jnp.float32)

    @functools.partial(
        pl.kernel,
        out_type=out_sds,
        mesh=mesh,
        scratch_types=[
            vmem_idx, vmem_idx, vmem_idx, vmem_idx, vmem_idx, vmem_idx,
            vmem_rows, vmem_rows, vmem_rows, vmem_rows, vmem_rows, vmem_rows,
            pltpu.SemaphoreType.DMA, pltpu.SemaphoreType.DMA,
            pltpu.SemaphoreType.DMA, pltpu.SemaphoreType.DMA,
            pltpu.SemaphoreType.DMA, pltpu.SemaphoreType.DMA,
        ],
    )
    def sck(wl_t, pos_t, hop_t, wl_idx, pos_idx, hop_idx, osum,
            i00, i01, i02, i10, i11, i12,
            g00, g01, g02, g10, g11, g12,
            semi0, semi1, semg0, semg1, semo0, semo1):
        tbl = (wl_t, pos_t, hop_t)
        idx = (wl_idx, pos_idx, hop_idx)
        ic = ((i00, i01, i02), (i10, i11, i12))
        gb = ((g00, g01, g02), (g10, g11, g12))
        semi = (semi0, semi1)
        semg = (semg0, semg1)
        semo = (semo0, semo1)

        wid = lax.axis_index("s") * mesh.num_cores + lax.axis_index("c")
        base = wid * rows_per_w

        def fire_idx(w, p):
            off = base + w * _WINDOW
            for k in range(3):
                pltpu.async_copy(idx[k].at[pl.ds(off, _WINDOW)], ic[p][k],
                                 semi[p])

        def drain_idx(p):
            for k in range(3):
                pltpu.make_async_copy(idx[k].at[pl.ds(base, _WINDOW)],
                                      ic[p][k], semi[p]).wait()

        def fire_gathers(p):
            for k in range(3):
                pltpu.async_copy(tbl[k].at[ic[p][k]], gb[p][k], semg[p])

        def drain_gathers(p):
            for k in range(3):
                pltpu.make_async_copy(tbl[k].at[pl.ds(0, _WINDOW)],
                                      gb[p][k], semg[p]).wait()

        def sum_bufs(p):
            a0, a1, a2 = gb[p]

            @pl.loop(0, _WINDOW)
            def _(r):
                for c in range(0, d, 16):
                    slc = (pl.ds(r, 1), pl.ds(c, 16))
                    a0.at[slc][...] = (
                        a0.at[slc][...] + a1.at[slc][...] + a2.at[slc][...])

        def fire_outputs(w, p):
            off = base + w * _WINDOW
            pltpu.async_copy(gb[p][0], osum.at[pl.ds(off, _WINDOW)], semo[p])

        def drain_outputs(p):
            pltpu.make_async_copy(tbl[0].at[pl.ds(0, _WINDOW)],
                                  gb[p][0], semo[p]).wait()

        fire_idx(0, 0)
        drain_idx(0)
        fire_gathers(0)
        fire_idx(1, 1)

        @pl.loop(0, n_win // 2)
        def _(j):
            for b in (0, 1):
                w = 2 * j + b
                nb = 1 - b

                @pl.when(w >= 1)
                def _():
                    drain_outputs(nb)

                @pl.when(w + 1 < n_win)
                def _():
                    drain_idx(nb)
                    fire_gathers(nb)

                drain_gathers(b)

                @pl.when(w + 2 < n_win)
                def _():
                    fire_idx(w + 2, b)

                sum_bufs(b)
                fire_outputs(w, b)

        drain_outputs((n_win - 1) % 2)

    return sck(wl_table, pos_table, hop_table, wl_i, pos_i, hop_i)


def _tc_body(raw_ref, g_ref, w_ref, b_ref, gamma_ref, beta_ref, o_ref):
    s, bt, d = raw_ref.shape
    x2 = raw_ref[...].reshape(s * bt, d)
    x = jnp.dot(x2, w_ref[...], preferred_element_type=jnp.float32)
    e = x + b_ref[...] + g_ref[...].reshape(s * bt, d)
    mean = jnp.mean(e, axis=-1, keepdims=True)
    c = e - mean
    var = jnp.mean(c * c, axis=-1, keepdims=True)
    o = c * lax.rsqrt(var + _EPS) * gamma_ref[...] + beta_ref[...]
    o_ref[...] = o.reshape(s, bt, d)


def _tc_body_alias(raw_ref, g_ref, w_ref, b_ref, gamma_ref, beta_ref,
                   prev_ref, o_ref):
    del prev_ref
    _tc_body(raw_ref, g_ref, w_ref, b_ref, gamma_ref, beta_ref, o_ref)


def _tc_fuse_chunk(raw_t, g3, w, b, gamma, beta, tile_b, chunk, spc, prev):
    s, batch, d = raw_t.shape
    grid = (batch // tile_b,)
    raw_spec = pl.BlockSpec((spc, tile_b, d), lambda i: (chunk, i, 0))
    g_spec = pl.BlockSpec((spc, tile_b, d), lambda i: (0, i, 0))
    full_spec = pl.BlockSpec((d, d), lambda i: (0, 0))
    vec_spec = pl.BlockSpec((1, d), lambda i: (0, 0))
    in_specs = [raw_spec, g_spec, full_spec, vec_spec, vec_spec, vec_spec]
    args = [raw_t, g3, w, b.reshape(1, d), gamma.reshape(1, d),
            beta.reshape(1, d)]
    kwargs = {}
    body = _tc_body
    if prev is not None:
        in_specs.append(pl.BlockSpec((spc, 8, d), lambda i: (0, 0, 0)))
        args.append(prev)
        kwargs["input_output_aliases"] = {6: 0}
        body = _tc_body_alias
    return pl.pallas_call(
        body,
        grid=grid,
        in_specs=in_specs,
        out_specs=raw_spec,
        out_shape=jax.ShapeDtypeStruct((s, batch, d), jnp.float32),
        compiler_params=pltpu.CompilerParams(
            dimension_semantics=("parallel",)),
        **kwargs,
    )(*args)


def kernel(raw_features, wl_role_ids, init_pos_ids, hop_dis_ids, W, b,
           wl_table, pos_table, hop_table, gamma, beta):
    batch, seq, x_size = raw_features.shape
    raw_t = raw_features.transpose(1, 0, 2)
    wl_t = wl_role_ids.T
    pos_t = init_pos_ids.T
    hop_t = hop_dis_ids.T
    spc = seq // _CHUNKS

    gs = []
    for c in range(_CHUNKS):
        sl = slice(c * spc, (c + 1) * spc)
        wl_i = wl_t[sl].reshape(-1).astype(jnp.int32)
        pos_i = pos_t[sl].reshape(-1).astype(jnp.int32)
        hop_i = hop_t[sl].reshape(-1).astype(jnp.int32)
        gs.append(_sc_gathersum(wl_table, pos_table, hop_table,
                                wl_i, pos_i, hop_i))

    out = None
    for c in range(_CHUNKS):
        g3 = gs[c].reshape(spc, batch, x_size)
        out = _tc_fuse_chunk(raw_t, g3, W, b, gamma, beta,
                             tile_b=512, chunk=c, spc=spc, prev=out)
    return out.transpose(1, 0, 2)
